# baseline (speedup 1.0000x reference)
import functools

import jax
import jax.numpy as jnp
from jax import lax
from jax.experimental import pallas as pl
from jax.experimental.pallas import tpu as pltpu
from jax.experimental.pallas import tpu_sc as plsc

_B = 16
_M = 8192
_H = 512
_W = 512
_NC = 2
_CHUNKS = (104, 96, 56)
_BUFROWS = (104, 96)
_UNROLL = 8


@functools.cache
def _build_scatter_kernel():
    mesh = plsc.VectorSubcoreMesh(core_axis_name="c", subcore_axis_name="s")

    @functools.partial(
        pl.kernel,
        out_type=jax.ShapeDtypeStruct((_B, _H, _W), jnp.float32),
        mesh=mesh,
        scratch_types=[
            pltpu.VMEM((_M,), jnp.int32),
            pltpu.VMEM((_M,), jnp.int32),
            pltpu.VMEM((_M,), jnp.float32),
            pltpu.VMEM((16,), jnp.int32),
            pltpu.VMEM((_BUFROWS[0], _W), jnp.float32),
            pltpu.VMEM((_BUFROWS[1], _W), jnp.float32),
            pltpu.SemaphoreType.DMA,
            pltpu.SemaphoreType.DMA,
            pltpu.SemaphoreType.DMA,
        ],
        compiler_params=pltpu.CompilerParams(needs_layout_passes=False),
    )
    def k(rows_hbm, cols_hbm, vals_hbm, nv_hbm, out_hbm,
          rows_v, cols_v, vals_v, nv_v, slab_a, slab_b, sem, sem_a, sem_b):
        wid = lax.axis_index("s") * _NC + lax.axis_index("c")
        b = wid // 2
        h = wid % 2
        cp_r = pltpu.async_copy(rows_hbm.at[b], rows_v, sem)
        cp_c = pltpu.async_copy(cols_hbm.at[b], cols_v, sem)
        cp_v = pltpu.async_copy(vals_hbm.at[b], vals_v, sem)
        cp_nv = pltpu.async_copy(nv_hbm, nv_v, sem)

        lane = lax.iota(jnp.int32, 16)
        zeros_f = jnp.zeros((16,), jnp.float32)

        def zero_slab(slab, nrows):
            @plsc.parallel_loop(0, nrows, 1, unroll=4)
            def _(i):
                for j in range(_W // 16):
                    slab[i, pl.ds(j * 16, 16)] = zeros_f

        zero_slab(slab_a, _CHUNKS[0])
        zero_slab(slab_b, _CHUNKS[1])
        cp_r.wait()
        cp_c.wait()
        cp_v.wait()
        cp_nv.wait()

        n = jnp.max(jnp.where(lane == b, nv_v[...], 0))
        half = h * 256

        def scan(targets, unroll):
            nsteps = (jnp.minimum((n + 15) // 16, _M // 16) + unroll - 1) // unroll

            def scatter_step(g, carry):
                staged = []
                for u in range(unroll):
                    base = (g * unroll + u) * 16
                    pos = lane + base
                    r = rows_v[pl.ds(base, 16)]
                    c = cols_v[pl.ds(base, 16)]
                    v = vals_v[pl.ds(base, 16)]
                    valid = pos < n
                    per_t = []
                    for _, lo, nrows in targets:
                        rr = r - lo
                        m = valid & (rr.astype(jnp.uint32) < nrows)
                        per_t.append((jnp.where(m, rr, 0), m))
                    staged.append((c, v, per_t))
                for c, v, per_t in staged:
                    for (slab, _, _), (rr, m) in zip(targets, per_t):
                        plsc.store_scatter(slab, [rr, c], v, mask=m)
                return carry

            lax.fori_loop(0, nsteps, scatter_step, 0)

        r0, r1, r2 = _CHUNKS
        scan([(slab_a, half, r0), (slab_b, half + r0, r1)], 4)
        cp_a = pltpu.async_copy(
            slab_a, out_hbm.at[b, pl.ds(half, r0)], sem_a)
        cp_b = pltpu.async_copy(
            slab_b, out_hbm.at[b, pl.ds(half + r0, r1)], sem_b)
        cp_a.wait()
        zero_slab(slab_a, r2)
        scan([(slab_a, half + r0 + r1, r2)], _UNROLL)
        cp_c = pltpu.async_copy(
            slab_a.at[pl.ds(0, r2)],
            out_hbm.at[b, pl.ds(half + r0 + r1, r2)], sem_a)
        cp_b.wait()
        cp_c.wait()

    return k


def kernel(indices, num_valid_coordinates, padded_features):
    rows = indices[..., 0]
    cols = indices[..., 1]
    vals = padded_features[..., 0]
    return _build_scatter_kernel()(rows, cols, vals, num_valid_coordinates)


# speedup vs baseline: 1.0748x; 1.0748x over previous
import functools

import jax
import jax.numpy as jnp
from jax import lax
from jax.experimental import pallas as pl
from jax.experimental.pallas import tpu as pltpu
from jax.experimental.pallas import tpu_sc as plsc

_B = 16
_M = 8192
_H = 512
_W = 512
_NC = 2
_CHUNKS = (104, 96, 56)
_BUFROWS = (104, 96)
_UNROLL = 16


@functools.cache
def _build_scatter_kernel():
    mesh = plsc.VectorSubcoreMesh(core_axis_name="c", subcore_axis_name="s")

    @functools.partial(
        pl.kernel,
        out_type=jax.ShapeDtypeStruct((_B, _H, _W), jnp.float32),
        mesh=mesh,
        scratch_types=[
            pltpu.VMEM((_M,), jnp.int32),
            pltpu.VMEM((_M,), jnp.int32),
            pltpu.VMEM((_M,), jnp.float32),
            pltpu.VMEM((16,), jnp.int32),
            pltpu.VMEM((_BUFROWS[0], _W), jnp.float32),
            pltpu.VMEM((_BUFROWS[1], _W), jnp.float32),
            pltpu.SemaphoreType.DMA,
            pltpu.SemaphoreType.DMA,
            pltpu.SemaphoreType.DMA,
        ],
        compiler_params=pltpu.CompilerParams(needs_layout_passes=False),
    )
    def k(rows_hbm, cols_hbm, vals_hbm, nv_hbm, out_hbm,
          rows_v, cols_v, vals_v, nv_v, slab_a, slab_b, sem, sem_a, sem_b):
        wid = lax.axis_index("s") * _NC + lax.axis_index("c")
        b = wid // 2
        h = wid % 2
        cp_r = pltpu.async_copy(rows_hbm.at[b], rows_v, sem)
        cp_c = pltpu.async_copy(cols_hbm.at[b], cols_v, sem)
        cp_v = pltpu.async_copy(vals_hbm.at[b], vals_v, sem)
        cp_nv = pltpu.async_copy(nv_hbm, nv_v, sem)

        lane = lax.iota(jnp.int32, 16)
        zeros_f = jnp.zeros((16,), jnp.float32)

        def zero_slab(slab, nrows):
            @plsc.parallel_loop(0, nrows, 1, unroll=4)
            def _(i):
                for j in range(_W // 16):
                    slab[i, pl.ds(j * 16, 16)] = zeros_f

        zero_slab(slab_a, _CHUNKS[0])
        zero_slab(slab_b, _CHUNKS[1])
        cp_r.wait()
        cp_c.wait()
        cp_v.wait()
        cp_nv.wait()

        n = jnp.max(jnp.where(lane == b, nv_v[...], 0))
        nsteps = (jnp.minimum((n + 15) // 16, _M // 16) + _UNROLL - 1) // _UNROLL

        half = h * 256
        row_off = 0
        out_cps = []
        for t, nrows in enumerate(_CHUNKS):
            slab = (slab_a, slab_b)[t % 2]
            osem = (sem_a, sem_b)[t % 2]
            lo = half + row_off
            if t >= 2:
                out_cps[t - 2].wait()
                zero_slab(slab, nrows)

            def scatter_step(g, carry, slab=slab, lo=lo, nrows=nrows):
                staged = []
                for u in range(_UNROLL):
                    base = (g * _UNROLL + u) * 16
                    pos = lane + base
                    r = rows_v[pl.ds(base, 16)]
                    c = cols_v[pl.ds(base, 16)]
                    v = vals_v[pl.ds(base, 16)]
                    rr = r - lo
                    m = (pos < n) & (rr.astype(jnp.uint32) < nrows)
                    staged.append((jnp.where(m, rr, 0), c, v, m))
                for rr, c, v, m in staged:
                    plsc.store_scatter(slab, [rr, c], v, mask=m)
                return carry

            lax.fori_loop(0, nsteps, scatter_step, 0)
            out_cps.append(pltpu.async_copy(
                slab.at[pl.ds(0, nrows)],
                out_hbm.at[b, pl.ds(lo, nrows)], osem))
            row_off += nrows
        out_cps[-2].wait()
        out_cps[-1].wait()

    return k


def kernel(indices, num_valid_coordinates, padded_features):
    rows = indices[..., 0]
    cols = indices[..., 1]
    vals = padded_features[..., 0]
    return _build_scatter_kernel()(rows, cols, vals, num_valid_coordinates)
